# Initial kernel scaffold; baseline (speedup 1.0000x reference)
import jax
import jax.numpy as jnp
from jax.experimental import pallas as pl


def kernel(x, edge_index, batch, params):
    raise NotImplementedError("write your pallas kernel here")


# speedup vs baseline: 7.6370x; 7.6370x over previous
import functools

import jax
import jax.numpy as jnp
from jax import lax
from jax.experimental import pallas as pl
from jax.experimental.pallas import tpu as pltpu
from jax.experimental.pallas import tpu_sc as plsc

N = 10000
E = 320000
F_IN = 128
HID = 128
CLS = 10
G = 128

NC = 2
NS = 16
NW = NC * NS
EPW = E // NW
CH = 125
NCHUNK = EPW // CH
NPAD = 10240
ROWS_PER_TILE = NPAD // NS

_EPS = 1e-5


def _sc_edge_agg_body(x_hbm, src_hbm, dst_hbm, zeros_hbm, out_hbm,
                      src_v, dst_v, buf_v, acc_sh, sem):
    c = lax.axis_index("c")
    s = lax.axis_index("s")
    wid = s * NC + c

    pltpu.sync_copy(src_hbm.at[wid], src_v)
    pltpu.sync_copy(dst_hbm.at[wid], dst_v)

    pltpu.sync_copy(zeros_hbm, acc_sh.at[pl.ds(s * ROWS_PER_TILE, ROWS_PER_TILE)])
    plsc.subcore_barrier()

    def body(j, carry):
        pltpu.async_copy(x_hbm.at[src_v.at[j]], buf_v, sem).wait()
        pltpu.sync_copy(buf_v, acc_sh.at[dst_v.at[j]], add=True)
        return carry

    lax.fori_loop(0, NCHUNK, body, 0)
    plsc.subcore_barrier()

    pltpu.sync_copy(acc_sh.at[pl.ds(s * ROWS_PER_TILE, ROWS_PER_TILE)],
                    out_hbm.at[c, pl.ds(s * ROWS_PER_TILE, ROWS_PER_TILE)])


@functools.cache
def _sc_edge_agg():
    mesh = plsc.VectorSubcoreMesh(core_axis_name="c", subcore_axis_name="s",
                                  num_cores=NC, num_subcores=NS)
    return pl.kernel(
        _sc_edge_agg_body,
        out_type=jax.ShapeDtypeStruct((NC, NPAD, HID), jnp.float32),
        mesh=mesh,
        scratch_types=[
            pltpu.VMEM((NCHUNK, CH), jnp.int32),
            pltpu.VMEM((NCHUNK, CH), jnp.int32),
            pltpu.VMEM((CH, HID), jnp.float32),
            pltpu.VMEM_SHARED((NPAD, HID), jnp.float32),
            pltpu.SemaphoreType.DMA,
        ],
    )


def _bn_norm(z, g, b):
    mu = jnp.mean(z, axis=0, keepdims=True)
    var = jnp.mean((z - mu) * (z - mu), axis=0, keepdims=True)
    return g * (z - mu) / jnp.sqrt(var + _EPS) + b


def _pre_body(x_ref, g_ref, b_ref, w_ref, o_ref):
    xb = _bn_norm(x_ref[...], g_ref[...], b_ref[...])
    o_ref[...] = jnp.maximum(jnp.dot(xb, w_ref[...],
                                     preferred_element_type=jnp.float32), 0.0)


_pre_call = pl.pallas_call(
    _pre_body,
    out_shape=jax.ShapeDtypeStruct((N, HID), jnp.float32),
)


def _gin_body(x_ref, a_ref, w1_ref, b1_ref, g_ref, b_ref, w2_ref, b2_ref, o_ref):
    h = x_ref[...] + a_ref[0, :N] + a_ref[1, :N]
    z = jnp.dot(h, w1_ref[...], preferred_element_type=jnp.float32) + b1_ref[...]
    r = jnp.maximum(_bn_norm(z, g_ref[...], b_ref[...]), 0.0)
    o_ref[...] = jnp.maximum(
        jnp.dot(r, w2_ref[...], preferred_element_type=jnp.float32) + b2_ref[...],
        0.0)


_gin_call = pl.pallas_call(
    _gin_body,
    out_shape=jax.ShapeDtypeStruct((N, HID), jnp.float32),
)


def _head_body(x_ref, batch_ref, gw_ref, gb_ref,
               fg_ref, fb_ref, lw_ref, lb_ref,
               hg_ref, hb_ref, cw_ref, cb_ref, o_ref):
    x = x_ref[...]
    gate = jnp.sum(x * gw_ref[...], axis=1, keepdims=True) + gb_ref[...]
    gids = lax.broadcasted_iota(jnp.int32, (1, G), 1)
    m = (batch_ref[...] == gids).astype(jnp.float32)
    neg = jnp.float32(-1e30)
    gm = jnp.max(jnp.where(m > 0, gate, neg), axis=0, keepdims=True)
    gm_b = jnp.sum(m * gm, axis=1, keepdims=True)
    e = jnp.exp(gate - gm_b)
    ssum = jnp.sum(m * e, axis=0, keepdims=True)
    denom = jnp.sum(m * ssum, axis=1, keepdims=True) + 1e-16
    w = e / denom
    hg = lax.dot_general(m * w, x, (((0,), (0,)), ((), ())),
                         preferred_element_type=jnp.float32)
    h = _bn_norm(hg, fg_ref[...], fb_ref[...])
    h = jnp.maximum(jnp.dot(h, lw_ref[...],
                            preferred_element_type=jnp.float32) + lb_ref[...], 0.0)
    h = _bn_norm(h, hg_ref[...], hb_ref[...])
    logits = jnp.dot(h, cw_ref[...],
                     preferred_element_type=jnp.float32) + cb_ref[...]
    mx = jnp.max(logits, axis=1, keepdims=True)
    lse = mx + jnp.log(jnp.sum(jnp.exp(logits - mx), axis=1, keepdims=True))
    o_ref[...] = logits - lse


_head_call = pl.pallas_call(
    _head_body,
    out_shape=jax.ShapeDtypeStruct((G, CLS), jnp.float32),
)


def kernel(x, edge_index, batch, params):
    p = params
    row = lambda v: v.reshape(1, -1)

    src = edge_index[0].reshape(NW, NCHUNK, CH)
    dst = edge_index[1].reshape(NW, NCHUNK, CH)
    zeros = jnp.zeros((ROWS_PER_TILE, HID), jnp.float32)
    batch2 = batch.reshape(N, 1)

    h = _pre_call(x, row(p['bn_feat_g']), row(p['bn_feat_b']), p['W_feat'])
    for gp in p['gins']:
        agg = _sc_edge_agg()(h, src, dst, zeros)
        h = _gin_call(h, agg, gp['W1'], row(gp['b1']), row(gp['bn_g']),
                      row(gp['bn_b']), gp['W2'], row(gp['b2']))
    return _head_call(h, batch2, row(p['gate_W'][:, 0]), row(p['gate_b']),
                      row(p['bn_fc0_g']), row(p['bn_fc0_b']),
                      p['lin0_W'], row(p['lin0_b']),
                      row(p['bn_hid_g']), row(p['bn_hid_b']),
                      p['cls_W'], row(p['cls_b']))
